# baseline (speedup 1.0000x reference)
import functools

import jax
import jax.numpy as jnp
from jax import lax
from jax.experimental import pallas as pl
from jax.experimental.pallas import tpu as pltpu
from jax.experimental.pallas import tpu_sc as plsc

_EPS = 1e-5
_K = 16
_LANES = 128


def _fps_body(m_per, seg,
              xs_ref, ys_ref, zs_ref, xs_s, ys_s, zs_s,
              ox_ref, oy_ref, oz_ref):
    nseg = xs_ref.shape[0]
    shp = xs_ref.shape[1:]
    flat = (lax.broadcasted_iota(jnp.int32, shp, 0) * _LANES +
            lax.broadcasted_iota(jnp.int32, shp, 1))

    init = []
    for s in range(nseg):
        xl = xs_s[s, 0]
        yl = ys_s[s, 0]
        zl = zs_s[s, 0]
        ox_ref[s, 0] = xl
        oy_ref[s, 0] = yl
        oz_ref[s, 0] = zl
        init.append((jnp.full(shp, 1e10, jnp.float32), xl, yl, zl))

    def body(i, carry):
        dists, ms = [], []
        for s in range(nseg):
            dist, xl, yl, zl = carry[s]
            dx = xs_ref[s] - xl
            dy = ys_ref[s] - yl
            dz = zs_ref[s] - zl
            d = dx * dx + dy * dy
            d = d + dz * dz
            dist = jnp.minimum(dist, d)
            dists.append(dist)
            ms.append(jnp.max(dist, axis=0, keepdims=True))
        mx4 = jnp.max(jnp.concatenate(ms, axis=0), axis=1, keepdims=True)
        ks = []
        for s in range(nseg):
            sel = jnp.where(dists[s] == mx4[s, 0], flat, jnp.int32(seg))
            ks.append(jnp.min(sel, axis=0, keepdims=True))
        nxt4 = jnp.min(jnp.concatenate(ks, axis=0), axis=1, keepdims=True)
        new = []
        for s in range(nseg):
            nxt = nxt4[s, 0]
            xl = xs_s[s, nxt]
            yl = ys_s[s, nxt]
            zl = zs_s[s, nxt]
            ox_ref[s, i] = xl
            oy_ref[s, i] = yl
            oz_ref[s, i] = zl
            new.append((dists[s], xl, yl, zl))
        return tuple(new)

    lax.fori_loop(1, m_per, body, tuple(init))


def _fps(xyz_seg, m_per):
    nseg, seg, _ = xyz_seg.shape
    xs = xyz_seg[:, :, 0]
    ys = xyz_seg[:, :, 1]
    zs = xyz_seg[:, :, 2]
    smem = pl.BlockSpec(memory_space=pltpu.SMEM)
    vmem = pl.BlockSpec(memory_space=pltpu.VMEM)
    ox, oy, oz = pl.pallas_call(
        functools.partial(_fps_body, m_per, seg),
        in_specs=[vmem, vmem, vmem, smem, smem, smem],
        out_shape=[jax.ShapeDtypeStruct((nseg, m_per), jnp.float32)] * 3,
        out_specs=[smem] * 3,
    )(xs.reshape(nseg, seg // _LANES, _LANES),
      ys.reshape(nseg, seg // _LANES, _LANES),
      zs.reshape(nseg, seg // _LANES, _LANES),
      xs, ys, zs)
    return jnp.stack([ox, oy, oz], axis=-1)


def _knn_body(seg, pts_ref, ctr_ref, out_ref):
    px = pts_ref[0, 0:1, :]
    py = pts_ref[0, 1:2, :]
    pz = pts_ref[0, 2:3, :]
    c = ctr_ref[0]
    cx = c[:, 0:1]
    cy = c[:, 1:2]
    cz = c[:, 2:3]
    dx = cx - px
    dy = cy - py
    dz = cz - pz
    d = dx * dx + dy * dy
    d = d + dz * dz
    iota = lax.broadcasted_iota(jnp.int32, d.shape, 1)
    cols = []
    for _ in range(_K):
        mn = jnp.min(d, axis=1, keepdims=True)
        sel = jnp.where(d == mn, iota, jnp.int32(seg))
        idx = jnp.min(sel, axis=1, keepdims=True)
        cols.append(idx)
        d = jnp.where(iota == idx, jnp.float32(jnp.inf), d)
    out_ref[0, 0] = jnp.concatenate(cols, axis=1)


def _knn(pts_t, ctr_pad, cblk):
    nseg, _, seg = pts_t.shape
    m_pad = ctr_pad.shape[1]
    nblk = m_pad // cblk
    return pl.pallas_call(
        functools.partial(_knn_body, seg),
        grid=(nseg, nblk),
        in_specs=[
            pl.BlockSpec((1, 3, seg), lambda s, b: (s, 0, 0)),
            pl.BlockSpec((1, cblk, 3), lambda s, b: (s, b, 0)),
        ],
        out_specs=pl.BlockSpec((1, 1, cblk, _K), lambda s, b: (s, b, 0, 0)),
        out_shape=jax.ShapeDtypeStruct((nseg, nblk, cblk, _K), jnp.int32),
    )(pts_t, ctr_pad)


def _lnmm_body(f_ref, wt_ref, g_ref, b_ref, o_ref):
    f = f_ref[...]
    mu = jnp.mean(f, axis=1, keepdims=True)
    var = jnp.mean((f - mu) ** 2, axis=1, keepdims=True)
    normed = (f - mu) / jnp.sqrt(var + _EPS) * g_ref[...] + b_ref[...]
    o_ref[...] = jnp.dot(normed, wt_ref[...],
                         preferred_element_type=jnp.float32,
                         precision=lax.Precision.HIGHEST)


def _lnmm(feats, w, gamma, beta, rblk):
    n, c_in = feats.shape
    c_out = w.shape[0]
    return pl.pallas_call(
        _lnmm_body,
        grid=(n // rblk,),
        in_specs=[
            pl.BlockSpec((rblk, c_in), lambda r: (r, 0)),
            pl.BlockSpec((c_in, c_out), lambda r: (0, 0)),
            pl.BlockSpec((1, c_in), lambda r: (0, 0)),
            pl.BlockSpec((1, c_in), lambda r: (0, 0)),
        ],
        out_specs=pl.BlockSpec((rblk, c_out), lambda r: (r, 0)),
        out_shape=jax.ShapeDtypeStruct((n, c_out), jnp.float32),
    )(feats, jnp.transpose(w), gamma.reshape(1, c_in), beta.reshape(1, c_in))


_SC_CORES = 2
_SC_SUBCORES = 16
_SC_NW = _SC_CORES * _SC_SUBCORES
_CC = 8


def _gather_max(table, idx_rows, m_sc, c_per_w):
    c_out = table.shape[1]
    ncol = c_out // 16
    nch = c_per_w // _CC
    mesh = plsc.VectorSubcoreMesh(core_axis_name="c", subcore_axis_name="s")

    @functools.partial(
        pl.kernel,
        mesh=mesh,
        out_type=jax.ShapeDtypeStruct((m_sc, c_out), jnp.float32),
        scratch_types=[
            pltpu.VMEM((c_per_w * _K,), jnp.int32),
            pltpu.VMEM((2, _CC * _K, c_out), jnp.float32),
            pltpu.VMEM((_CC, c_out), jnp.float32),
            pltpu.SemaphoreType.DMA,
            pltpu.SemaphoreType.DMA,
        ],
    )
    def k(table_hbm, idx_hbm, out_hbm, idx_v, rows_v, out_v, sem0, sem1):
        wid = lax.axis_index("s") * _SC_CORES + lax.axis_index("c")
        cbase = wid * c_per_w
        pltpu.sync_copy(idx_hbm.at[pl.ds(cbase * _K, c_per_w * _K)], idx_v)
        sems = [sem0, sem1]
        nrow = _CC * _K
        pending = pltpu.async_copy(
            table_hbm.at[idx_v.at[pl.ds(0, nrow)]], rows_v.at[0], sems[0])
        for ch in range(nch):
            slot = ch & 1
            if ch + 1 < nch:
                nxt = pltpu.async_copy(
                    table_hbm.at[idx_v.at[pl.ds((ch + 1) * nrow, nrow)]],
                    rows_v.at[1 - slot], sems[1 - slot])
            pending.wait()

            def per_center(g, carry2, slot=slot):
                base = g * _K
                for c in range(ncol):
                    sl = pl.ds(c * 16, 16)
                    acc = rows_v[slot, base, sl]
                    for r in range(1, _K):
                        acc = jnp.maximum(acc, rows_v[slot, base + r, sl])
                    out_v[g, sl] = acc
                return carry2

            lax.fori_loop(0, _CC, per_center, 0)
            pltpu.sync_copy(out_v, out_hbm.at[pl.ds(cbase + ch * _CC, _CC)])
            if ch + 1 < nch:
                pending = nxt

    return k(table, idx_rows)


def kernel(xyz, feats, ln_gamma, ln_beta, W, offset):
    n = xyz.shape[0]
    b = offset.shape[0]
    seg = n // b
    m_per = int(seg * 0.25) + 1
    m = b * m_per

    cblk = 128
    m_pad = ((m_per + cblk - 1) // cblk) * cblk

    unit = _SC_NW * _CC
    m_sc = ((m + unit - 1) // unit) * unit
    c_per_w = m_sc // _SC_NW

    xyz_seg = xyz.reshape(b, seg, 3)

    nxyz = _fps(xyz_seg, m_per)

    pts_t = jnp.transpose(xyz_seg, (0, 2, 1))
    ctr_pad = jnp.concatenate(
        [nxyz, jnp.zeros((b, m_pad - m_per, 3), jnp.float32)], axis=1)
    knn = _knn(pts_t, ctr_pad, cblk)
    knn = knn.reshape(b, m_pad, _K)[:, :m_per]
    knn = knn + (jnp.arange(b, dtype=jnp.int32) * seg)[:, None, None]
    idx_rows = jnp.concatenate(
        [knn.reshape(m * _K),
         jnp.zeros(((m_sc - m) * _K,), jnp.int32)])

    table = _lnmm(feats, W, ln_gamma, ln_beta, rblk=1024)

    out = _gather_max(table, idx_rows, m_sc, c_per_w)[:m]

    n_xyz = nxyz.reshape(m, 3)
    n_offset = (jnp.arange(1, b + 1, dtype=jnp.int32) * m_per).astype(jnp.int32)
    return (n_xyz, out, n_offset)


# speedup vs baseline: 1.1800x; 1.1800x over previous
import functools

import jax
import jax.numpy as jnp
from jax import lax
from jax.experimental import pallas as pl
from jax.experimental.pallas import tpu as pltpu
from jax.experimental.pallas import tpu_sc as plsc

_EPS = 1e-5
_K = 16
_LANES = 128


def _fps_body(m_per, seg,
              xs_ref, ys_ref, zs_ref, xs_s, ys_s, zs_s,
              ox_ref, oy_ref, oz_ref):
    nseg = xs_ref.shape[0]
    shp = xs_ref.shape[1:]
    flat = (lax.broadcasted_iota(jnp.int32, shp, 0) * _LANES +
            lax.broadcasted_iota(jnp.int32, shp, 1))

    init = []
    for s in range(nseg):
        xl = xs_s[s, 0]
        yl = ys_s[s, 0]
        zl = zs_s[s, 0]
        ox_ref[s, 0] = xl
        oy_ref[s, 0] = yl
        oz_ref[s, 0] = zl
        init.append((jnp.full(shp, 1e10, jnp.float32), xl, yl, zl))

    def body(i, carry):
        dists, ms = [], []
        for s in range(nseg):
            dist, xl, yl, zl = carry[s]
            dx = xs_ref[s] - xl
            dy = ys_ref[s] - yl
            dz = zs_ref[s] - zl
            d = dx * dx + dy * dy
            d = d + dz * dz
            dist = jnp.minimum(dist, d)
            dists.append(dist)
            ms.append(jnp.max(dist, axis=0, keepdims=True))
        mx4 = jnp.max(jnp.concatenate(ms, axis=0), axis=1, keepdims=True)
        ks = []
        for s in range(nseg):
            sel = jnp.where(dists[s] == mx4[s, 0], flat, jnp.int32(seg))
            ks.append(jnp.min(sel, axis=0, keepdims=True))
        nxt4 = jnp.min(jnp.concatenate(ks, axis=0), axis=1, keepdims=True)
        new = []
        for s in range(nseg):
            nxt = nxt4[s, 0]
            xl = xs_s[s, nxt]
            yl = ys_s[s, nxt]
            zl = zs_s[s, nxt]
            ox_ref[s, i] = xl
            oy_ref[s, i] = yl
            oz_ref[s, i] = zl
            new.append((dists[s], xl, yl, zl))
        return tuple(new)

    lax.fori_loop(1, m_per, body, tuple(init))


def _fps(xyz_seg, m_per, m_pad):
    nseg, seg, _ = xyz_seg.shape
    xs = xyz_seg[:, :, 0]
    ys = xyz_seg[:, :, 1]
    zs = xyz_seg[:, :, 2]
    smem = pl.BlockSpec(memory_space=pltpu.SMEM)
    vmem = pl.BlockSpec(memory_space=pltpu.VMEM)
    ox, oy, oz = pl.pallas_call(
        functools.partial(_fps_body, m_per, seg),
        in_specs=[vmem, vmem, vmem, smem, smem, smem],
        out_shape=[jax.ShapeDtypeStruct((nseg, m_pad), jnp.float32)] * 3,
        out_specs=[smem] * 3,
    )(xs.reshape(nseg, seg // _LANES, _LANES),
      ys.reshape(nseg, seg // _LANES, _LANES),
      zs.reshape(nseg, seg // _LANES, _LANES),
      xs, ys, zs)
    return jnp.stack([ox, oy, oz], axis=-1)


def _knn_body(seg, pts_ref, ctr_ref, out_ref):
    base = pl.program_id(0) * seg
    px = pts_ref[0, 0:1, :]
    py = pts_ref[0, 1:2, :]
    pz = pts_ref[0, 2:3, :]
    c = ctr_ref[0]
    cx = c[:, 0:1]
    cy = c[:, 1:2]
    cz = c[:, 2:3]
    dx = cx - px
    dy = cy - py
    dz = cz - pz
    d = dx * dx + dy * dy
    d = d + dz * dz
    iota = lax.broadcasted_iota(jnp.int32, d.shape, 1)
    cols = []
    for _ in range(_K):
        mn = jnp.min(d, axis=1, keepdims=True)
        sel = jnp.where(d == mn, iota, jnp.int32(seg))
        idx = jnp.min(sel, axis=1, keepdims=True)
        cols.append(jnp.minimum(idx, seg - 1) + base)
        d = jnp.where(iota == idx, jnp.float32(jnp.inf), d)
    out_ref[0, 0] = jnp.concatenate(cols, axis=1)


def _knn(pts_t, ctr_pad, cblk):
    nseg, _, seg = pts_t.shape
    m_pad = ctr_pad.shape[1]
    nblk = m_pad // cblk
    return pl.pallas_call(
        functools.partial(_knn_body, seg),
        grid=(nseg, nblk),
        in_specs=[
            pl.BlockSpec((1, 3, seg), lambda s, b: (s, 0, 0)),
            pl.BlockSpec((1, cblk, 3), lambda s, b: (s, b, 0)),
        ],
        out_specs=pl.BlockSpec((1, 1, cblk, _K), lambda s, b: (s, b, 0, 0)),
        out_shape=jax.ShapeDtypeStruct((nseg, nblk, cblk, _K), jnp.int32),
    )(pts_t, ctr_pad)


def _lnmm_body(f_ref, wt_ref, g_ref, b_ref, o_ref):
    f = f_ref[...]
    mu = jnp.mean(f, axis=1, keepdims=True)
    var = jnp.mean((f - mu) ** 2, axis=1, keepdims=True)
    normed = (f - mu) / jnp.sqrt(var + _EPS) * g_ref[...] + b_ref[...]
    o_ref[...] = jnp.dot(normed, wt_ref[...],
                         preferred_element_type=jnp.float32,
                         precision=lax.Precision.HIGHEST)


def _lnmm(feats, w, gamma, beta, rblk):
    n, c_in = feats.shape
    c_out = w.shape[0]
    return pl.pallas_call(
        _lnmm_body,
        grid=(n // rblk,),
        in_specs=[
            pl.BlockSpec((rblk, c_in), lambda r: (r, 0)),
            pl.BlockSpec((c_in, c_out), lambda r: (0, 0)),
            pl.BlockSpec((1, c_in), lambda r: (0, 0)),
            pl.BlockSpec((1, c_in), lambda r: (0, 0)),
        ],
        out_specs=pl.BlockSpec((rblk, c_out), lambda r: (r, 0)),
        out_shape=jax.ShapeDtypeStruct((n, c_out), jnp.float32),
    )(feats, jnp.transpose(w), gamma.reshape(1, c_in), beta.reshape(1, c_in))


_SC_CORES = 2
_SC_SUBCORES = 16
_SC_NW = _SC_CORES * _SC_SUBCORES
_CC = 8
_DEPTH = 4


def _gather_max(table, idx_rows, m_sc, c_per_w):
    c_out = table.shape[1]
    ncol = c_out // 16
    nch = c_per_w // _CC
    mesh = plsc.VectorSubcoreMesh(core_axis_name="c", subcore_axis_name="s")

    @functools.partial(
        pl.kernel,
        mesh=mesh,
        out_type=jax.ShapeDtypeStruct((m_sc, c_out), jnp.float32),
        scratch_types=[
            pltpu.VMEM((c_per_w * _K,), jnp.int32),
            pltpu.VMEM((_DEPTH, _CC * _K, c_out), jnp.float32),
            pltpu.VMEM((_CC, c_out), jnp.float32),
            pltpu.SemaphoreType.DMA,
            pltpu.SemaphoreType.DMA,
            pltpu.SemaphoreType.DMA,
            pltpu.SemaphoreType.DMA,
        ],
    )
    def k(table_hbm, idx_hbm, out_hbm, idx_v, rows_v, out_v, *sems):
        wid = lax.axis_index("s") * _SC_CORES + lax.axis_index("c")
        cbase = wid * c_per_w
        pltpu.sync_copy(idx_hbm.at[pl.ds(cbase * _K, c_per_w * _K)], idx_v)
        nrow = _CC * _K

        def fire(ch):
            slot = ch % _DEPTH
            return pltpu.async_copy(
                table_hbm.at[idx_v.at[pl.ds(ch * nrow, nrow)]],
                rows_v.at[slot], sems[slot])

        copies = {ch: fire(ch) for ch in range(min(_DEPTH, nch))}
        for ch in range(nch):
            slot = ch % _DEPTH
            copies[ch].wait()

            def per_center(g, carry2, slot=slot):
                base = g * _K
                for c in range(ncol):
                    sl = pl.ds(c * 16, 16)
                    acc = rows_v[slot, base, sl]
                    for r in range(1, _K):
                        acc = jnp.maximum(acc, rows_v[slot, base + r, sl])
                    out_v[g, sl] = acc
                return carry2

            lax.fori_loop(0, _CC, per_center, 0)
            pltpu.sync_copy(out_v, out_hbm.at[pl.ds(cbase + ch * _CC, _CC)])
            if ch + _DEPTH < nch:
                copies[ch + _DEPTH] = fire(ch + _DEPTH)

    return k(table, idx_rows)


def kernel(xyz, feats, ln_gamma, ln_beta, W, offset):
    n = xyz.shape[0]
    b = offset.shape[0]
    seg = n // b
    m_per = int(seg * 0.25) + 1
    m = b * m_per

    cblk = 128
    m_pad = ((m_per + cblk - 1) // cblk) * cblk

    m_sc = b * m_pad
    c_per_w = m_sc // _SC_NW

    xyz_seg = xyz.reshape(b, seg, 3)

    nxyz_pad = _fps(xyz_seg, m_per, m_pad)

    pts_t = jnp.transpose(xyz_seg, (0, 2, 1))
    knn = _knn(pts_t, nxyz_pad, cblk)
    idx_rows = knn.reshape(m_sc * _K)

    table = _lnmm(feats, W, ln_gamma, ln_beta, rblk=1024)

    out = _gather_max(table, idx_rows, m_sc, c_per_w)
    out = out.reshape(b, m_pad, -1)[:, :m_per].reshape(m, -1)

    n_xyz = nxyz_pad[:, :m_per].reshape(m, 3)
    n_offset = (jnp.arange(1, b + 1, dtype=jnp.int32) * m_per).astype(jnp.int32)
    return (n_xyz, out, n_offset)
